# baseline (speedup 1.0000x reference)
import functools

import jax
import jax.numpy as jnp
from jax import lax
from jax.experimental import pallas as pl
from jax.experimental.pallas import tpu as pltpu
from jax.experimental.pallas import tpu_sc as plsc

N = 10000
DF = 128
H = 64
E = 160000
NS = 16
NC = 2
CHUNK = 128
NCH = 80
EPAD = NS * NCH * CHUNK
NROW = 10016
RPT = NROW // NS
RB = 2000
GRID = N // RB

_mesh = plsc.VectorSubcoreMesh(core_axis_name="c", subcore_axis_name="s")


NBUF = 2
RPT_T = N // NS


def _sc_body(with_deg, *refs):
    if with_deg:
        (t0, t1, t2, t3, s0, s1, s2, s3, d0, d1, d2, d3, z64, z16, ones_h,
         a0, a1, a2, a3, g0, g1, g2, g3,
         src_v, dst_v, rows_v, ones_v, acc, table, dacc, gsem) = refs
        gouts = (g0, g1, g2, g3)
    else:
        (t0, t1, t2, t3, s0, s1, s2, s3, d0, d1, d2, d3, z64,
         a0, a1, a2, a3,
         src_v, dst_v, rows_v, acc, table, gsem) = refs
    ts = (t0, t1, t2, t3)
    srcs = (s0, s1, s2, s3)
    dsts = (d0, d1, d2, d3)
    aouts = (a0, a1, a2, a3)

    c = lax.axis_index("c")
    s = lax.axis_index("s")
    base = s * RPT
    sl = pl.ds(base, RPT)
    tsl = pl.ds(s * RPT_T, RPT_T)

    pltpu.sync_copy(z64, acc.at[sl])
    if with_deg:
        pltpu.sync_copy(z16, dacc.at[sl])
        pltpu.sync_copy(ones_h, ones_v)
    plsc.subcore_barrier()

    for core_id in range(NC):
        @pl.when(c == core_id)
        def _():
            for j in range(2):
                et = 2 * core_id + j
                pltpu.sync_copy(ts[et].at[tsl], table.at[tsl])
                pltpu.sync_copy(srcs[et].at[s], src_v)
                pltpu.sync_copy(dsts[et].at[s], dst_v)
                plsc.subcore_barrier()

                for b in range(NBUF):
                    pltpu.async_copy(table.at[src_v.at[b]], rows_v.at[b],
                                     gsem.at[b])

                def group(go, carry):
                    for b in range(NBUF):
                        ch = go * NBUF + b
                        pltpu.make_async_copy(
                            table.at[src_v.at[ch]], rows_v.at[b],
                            gsem.at[b]).wait()
                        pltpu.sync_copy(rows_v.at[b],
                                        acc.at[dst_v.at[ch]], add=True)
                        if with_deg:
                            pltpu.sync_copy(ones_v,
                                            dacc.at[dst_v.at[ch]], add=True)
                        nxt = ch + NBUF

                        @pl.when(nxt < NCH)
                        def _():
                            pltpu.async_copy(table.at[src_v.at[nxt]],
                                             rows_v.at[b], gsem.at[b])
                    return carry

                lax.fori_loop(0, NCH // NBUF, group, 0)

                plsc.subcore_barrier()
                pltpu.sync_copy(acc.at[sl], aouts[et].at[sl])
                if with_deg:
                    pltpu.sync_copy(dacc.at[sl], gouts[et].at[sl])
                if j == 0:
                    pltpu.sync_copy(z64, acc.at[sl])
                    if with_deg:
                        pltpu.sync_copy(z16, dacc.at[sl])
                    plsc.subcore_barrier()


def _make_sc(with_deg):
    a_ty = jax.ShapeDtypeStruct((NROW, H), jnp.float32)
    g_ty = jax.ShapeDtypeStruct((NROW, 16), jnp.float32)
    scratch = [
        pltpu.VMEM((NCH, CHUNK), jnp.int32),
        pltpu.VMEM((NCH, CHUNK), jnp.int32),
        pltpu.VMEM((NBUF, CHUNK, H), jnp.float32),
    ]
    if with_deg:
        scratch.append(pltpu.VMEM((CHUNK, 16), jnp.float32))
    scratch.append(pltpu.VMEM_SHARED((NROW, H), jnp.float32))
    scratch.append(pltpu.VMEM_SHARED((N, H), jnp.float32))
    if with_deg:
        scratch.append(pltpu.VMEM_SHARED((NROW, 16), jnp.float32))
    scratch.append(pltpu.SemaphoreType.DMA((NBUF,)))
    return pl.kernel(
        functools.partial(_sc_body, with_deg),
        out_type=(a_ty,) * 4 + ((g_ty,) * 4 if with_deg else ()),
        mesh=_mesh, scratch_types=scratch,
        compiler_params=pltpu.CompilerParams(use_tc_tiling_on_sc=False))


_sc_agg_deg = _make_sc(True)
_sc_agg = _make_sc(False)


def _tc0_body(x0, x1, w, o0, o1, o2, o3):
    for k, (xr, orf) in enumerate(((x0, o0), (x1, o1), (x0, o2), (x1, o3))):
        orf[...] = jnp.dot(xr[...], w[k], preferred_element_type=jnp.float32)


_tc0 = pl.pallas_call(
    _tc0_body,
    grid=(GRID,),
    in_specs=[
        pl.BlockSpec((RB, DF), lambda i: (i, 0)),
        pl.BlockSpec((RB, DF), lambda i: (i, 0)),
        pl.BlockSpec((4, DF, H), lambda i: (0, 0, 0)),
    ],
    out_specs=[pl.BlockSpec((RB, H), lambda i: (i, 0))] * 4,
    out_shape=[jax.ShapeDtypeStruct((N, H), jnp.float32)] * 4,
)


def _norm_pair(a0, a1, g0, g1):
    r0 = a0[...] / jnp.maximum(g0[:, 0:1], 1.0)
    r1 = a1[...] / jnp.maximum(g1[:, 0:1], 1.0)
    return r0 + r1


def _tcmid_body(a0, a1, a2, a3, g0, g1, g2, g3, w,
                p0, p1, o0, o1, o2, o3):
    s0 = _norm_pair(a0, a1, g0, g1)
    s1 = _norm_pair(a2, a3, g2, g3)
    p0[...] = jnp.maximum(s0, 0.0)
    p1[...] = jnp.maximum(s1, 0.0)
    for k, (pr, orf) in enumerate(((p0, o0), (p1, o1), (p0, o2), (p1, o3))):
        orf[...] = jnp.dot(pr[...], w[k], preferred_element_type=jnp.float32)


_tcmid = pl.pallas_call(
    _tcmid_body,
    grid=(GRID,),
    in_specs=[pl.BlockSpec((RB, H), lambda i: (i, 0))] * 4
    + [pl.BlockSpec((RB, 16), lambda i: (i, 0))] * 4
    + [pl.BlockSpec((4, H, H), lambda i: (0, 0, 0))],
    out_specs=[pl.BlockSpec((RB, H), lambda i: (i, 0))] * 6,
    out_shape=[jax.ShapeDtypeStruct((N, H), jnp.float32)] * 6,
)


def _tcfin_body(a0, a1, a2, a3, g0, g1, g2, g3, h0, h1, q0, q1, att, out):
    emb0 = _norm_pair(a0, a1, g0, g1) + h0[...]
    emb1 = _norm_pair(a2, a3, g2, g3) + h1[...]
    out[0, :, 0:H] = h0[...] * att[0]
    out[0, :, H:2 * H] = q0[...] * att[1]
    out[0, :, 2 * H:3 * H] = emb0 * att[2]
    out[1, :, 0:H] = h1[...] * att[0]
    out[1, :, H:2 * H] = q1[...] * att[1]
    out[1, :, 2 * H:3 * H] = emb1 * att[2]


_tcfin = pl.pallas_call(
    _tcfin_body,
    grid=(GRID,),
    in_specs=[pl.BlockSpec((RB, H), lambda i: (i, 0))] * 4
    + [pl.BlockSpec((RB, 16), lambda i: (i, 0))] * 4
    + [pl.BlockSpec((RB, H), lambda i: (i, 0))] * 4
    + [pl.BlockSpec(memory_space=pltpu.SMEM)],
    out_specs=pl.BlockSpec((2, RB, 3 * H), lambda i: (0, i, 0)),
    out_shape=jax.ShapeDtypeStruct((2, N, 3 * H), jnp.float32),
)


def _prep_edges(e):
    pad = EPAD - E
    src = jnp.concatenate([e[0], jnp.zeros((pad,), jnp.int32)])
    dst = jnp.concatenate([e[1], jnp.full((pad,), N, jnp.int32)])
    return src.reshape(NS, NCH, CHUNK), dst.reshape(NS, NCH, CHUNK)


def kernel(x0, x1, e00, e01, e10, e11, W0, Wh, att):
    srcs, dsts = zip(*(_prep_edges(e) for e in (e00, e01, e10, e11)))
    z64 = jnp.zeros((RPT, H), jnp.float32)
    z16 = jnp.zeros((RPT, 16), jnp.float32)
    ones = jnp.ones((CHUNK, 16), jnp.float32)

    t = _tc0(x0, x1, W0)
    res = _sc_agg_deg(*t, *srcs, *dsts, z64, z16, ones)
    a, g = res[:4], res[4:]
    h0, h1, *t = _tcmid(*a, *g, Wh[0])
    a = _sc_agg(*t, *srcs, *dsts, z64)
    q0, q1, *t = _tcmid(*a, *g, Wh[1])
    a = _sc_agg(*t, *srcs, *dsts, z64)
    p0, p1, *t = _tcmid(*a, *g, Wh[2])
    a = _sc_agg(*t, *srcs, *dsts, z64)
    _, _, *t = _tcmid(*a, *g, Wh[3])
    a = _sc_agg(*t, *srcs, *dsts, z64)
    return _tcfin(*a, *g, h0, h1, q0, q1, att)


# speedup vs baseline: 1.0448x; 1.0448x over previous
import functools

import jax
import jax.numpy as jnp
from jax import lax
from jax.experimental import pallas as pl
from jax.experimental.pallas import tpu as pltpu
from jax.experimental.pallas import tpu_sc as plsc

N = 10000
DF = 128
H = 64
E = 160000
NS = 16
NC = 2
CHUNK = 256
NCH = 40
EPAD = NS * NCH * CHUNK
NROW = 10240
RPT = NROW // NS
RB = 2000
GRID = N // RB

_mesh = plsc.VectorSubcoreMesh(core_axis_name="c", subcore_axis_name="s")


NBUF = 2
RPT_T = N // NS


def _sc_body(*refs):
    (t0, t1, t2, t3, s0, s1, s2, s3, d0, d1, d2, d3, z64,
     a0, a1, a2, a3,
     src_v, dst_v, rows_v, acc, table, gsem) = refs
    ts = (t0, t1, t2, t3)
    srcs = (s0, s1, s2, s3)
    dsts = (d0, d1, d2, d3)
    aouts = (a0, a1, a2, a3)

    c = lax.axis_index("c")
    s = lax.axis_index("s")
    base = s * RPT
    sl = pl.ds(base, RPT)
    tsl = pl.ds(s * RPT_T, RPT_T)

    pltpu.sync_copy(z64, acc.at[sl])
    plsc.subcore_barrier()

    for core_id in range(NC):
        @pl.when(c == core_id)
        def _():
            for j in range(2):
                et = 2 * core_id + j
                pltpu.sync_copy(ts[et].at[tsl], table.at[tsl])
                plsc.subcore_barrier()

                for half in range(2):
                    hb = half * (NCH // 2)
                    pltpu.sync_copy(
                        srcs[et].at[s].at[pl.ds(hb, NCH // 2)], src_v)
                    pltpu.sync_copy(
                        dsts[et].at[s].at[pl.ds(hb, NCH // 2)], dst_v)

                    for b in range(NBUF):
                        pltpu.async_copy(table.at[src_v.at[b]],
                                         rows_v.at[b], gsem.at[b])

                    def group(go, carry):
                        for b in range(NBUF):
                            ch = go * NBUF + b
                            pltpu.make_async_copy(
                                table.at[src_v.at[ch]], rows_v.at[b],
                                gsem.at[b]).wait()
                            pltpu.sync_copy(rows_v.at[b],
                                            acc.at[dst_v.at[ch]], add=True)
                            nxt = ch + NBUF

                            @pl.when(nxt < NCH // 2)
                            def _():
                                pltpu.async_copy(table.at[src_v.at[nxt]],
                                                 rows_v.at[b], gsem.at[b])
                        return carry

                    lax.fori_loop(0, NCH // 2 // NBUF, group, 0)

                plsc.subcore_barrier()
                pltpu.sync_copy(acc.at[sl], aouts[et].at[sl])
                if j == 0:
                    pltpu.sync_copy(z64, acc.at[sl])
                    plsc.subcore_barrier()


_sc_agg = pl.kernel(
    _sc_body,
    out_type=(jax.ShapeDtypeStruct((NROW, H), jnp.float32),) * 4,
    mesh=_mesh,
    scratch_types=[
        pltpu.VMEM((NCH // 2, CHUNK), jnp.int32),
        pltpu.VMEM((NCH // 2, CHUNK), jnp.int32),
        pltpu.VMEM((NBUF, CHUNK, H), jnp.float32),
        pltpu.VMEM_SHARED((NROW, H), jnp.float32),
        pltpu.VMEM_SHARED((N, H), jnp.float32),
        pltpu.SemaphoreType.DMA((NBUF,)),
    ],
    compiler_params=pltpu.CompilerParams(use_tc_tiling_on_sc=False))


def _sc_deg_body(*refs):
    (d0, d1, d2, d3, z16, ones_h, g0, g1, g2, g3,
     dst_v, ones_v, dacc) = refs
    dsts = (d0, d1, d2, d3)
    gouts = (g0, g1, g2, g3)

    c = lax.axis_index("c")
    s = lax.axis_index("s")
    base = s * RPT

    for j in range(2):
        pltpu.sync_copy(z16, dacc.at[j].at[pl.ds(base, RPT)])
    pltpu.sync_copy(ones_h, ones_v)
    plsc.subcore_barrier()

    for core_id in range(NC):
        @pl.when(c == core_id)
        def _():
            for j in range(2):
                et = 2 * core_id + j
                pltpu.sync_copy(dsts[et].at[s], dst_v)

                def chunk(ch, carry):
                    pltpu.sync_copy(ones_v, dacc.at[j].at[dst_v.at[ch]],
                                    add=True)
                    return carry

                lax.fori_loop(0, NCH, chunk, 0)

    plsc.subcore_barrier()

    for core_id in range(NC):
        @pl.when(c == core_id)
        def _():
            for j in range(2):
                et = 2 * core_id + j
                sl = pl.ds(base, RPT)
                pltpu.sync_copy(dacc.at[j].at[sl], gouts[et].at[sl])


_sc_deg = pl.kernel(
    _sc_deg_body,
    out_type=(jax.ShapeDtypeStruct((NROW, 16), jnp.float32),) * 4,
    mesh=_mesh,
    scratch_types=[
        pltpu.VMEM((NCH, CHUNK), jnp.int32),
        pltpu.VMEM((CHUNK, 16), jnp.float32),
        pltpu.VMEM_SHARED((2, NROW, 16), jnp.float32),
    ],
    compiler_params=pltpu.CompilerParams(use_tc_tiling_on_sc=False))


def _tc0_body(x0, x1, w, o0, o1, o2, o3):
    for k, (xr, orf) in enumerate(((x0, o0), (x1, o1), (x0, o2), (x1, o3))):
        orf[...] = jnp.dot(xr[...], w[k], preferred_element_type=jnp.float32)


_tc0 = pl.pallas_call(
    _tc0_body,
    grid=(GRID,),
    in_specs=[
        pl.BlockSpec((RB, DF), lambda i: (i, 0)),
        pl.BlockSpec((RB, DF), lambda i: (i, 0)),
        pl.BlockSpec((4, DF, H), lambda i: (0, 0, 0)),
    ],
    out_specs=[pl.BlockSpec((RB, H), lambda i: (i, 0))] * 4,
    out_shape=[jax.ShapeDtypeStruct((N, H), jnp.float32)] * 4,
)


def _norm_pair(a0, a1, g0, g1):
    r0 = a0[...] / jnp.maximum(g0[:, 0:1], 1.0)
    r1 = a1[...] / jnp.maximum(g1[:, 0:1], 1.0)
    return r0 + r1


def _tcmid_body(a0, a1, a2, a3, g0, g1, g2, g3, w,
                p0, p1, o0, o1, o2, o3):
    s0 = _norm_pair(a0, a1, g0, g1)
    s1 = _norm_pair(a2, a3, g2, g3)
    p0[...] = jnp.maximum(s0, 0.0)
    p1[...] = jnp.maximum(s1, 0.0)
    for k, (pr, orf) in enumerate(((p0, o0), (p1, o1), (p0, o2), (p1, o3))):
        orf[...] = jnp.dot(pr[...], w[k], preferred_element_type=jnp.float32)


_tcmid = pl.pallas_call(
    _tcmid_body,
    grid=(GRID,),
    in_specs=[pl.BlockSpec((RB, H), lambda i: (i, 0))] * 4
    + [pl.BlockSpec((RB, 16), lambda i: (i, 0))] * 4
    + [pl.BlockSpec((4, H, H), lambda i: (0, 0, 0))],
    out_specs=[pl.BlockSpec((RB, H), lambda i: (i, 0))] * 6,
    out_shape=[jax.ShapeDtypeStruct((N, H), jnp.float32)] * 6,
)


def _tcfin_body(a0, a1, a2, a3, g0, g1, g2, g3, h0, h1, q0, q1, att, out):
    emb0 = _norm_pair(a0, a1, g0, g1) + h0[...]
    emb1 = _norm_pair(a2, a3, g2, g3) + h1[...]
    out[0, :, 0:H] = h0[...] * att[0]
    out[0, :, H:2 * H] = q0[...] * att[1]
    out[0, :, 2 * H:3 * H] = emb0 * att[2]
    out[1, :, 0:H] = h1[...] * att[0]
    out[1, :, H:2 * H] = q1[...] * att[1]
    out[1, :, 2 * H:3 * H] = emb1 * att[2]


_tcfin = pl.pallas_call(
    _tcfin_body,
    grid=(GRID,),
    in_specs=[pl.BlockSpec((RB, H), lambda i: (i, 0))] * 4
    + [pl.BlockSpec((RB, 16), lambda i: (i, 0))] * 4
    + [pl.BlockSpec((RB, H), lambda i: (i, 0))] * 4
    + [pl.BlockSpec(memory_space=pltpu.SMEM)],
    out_specs=pl.BlockSpec((2, RB, 3 * H), lambda i: (0, i, 0)),
    out_shape=jax.ShapeDtypeStruct((2, N, 3 * H), jnp.float32),
)


def _prep_edges(e):
    pad = EPAD - E
    src = jnp.concatenate([e[0], jnp.zeros((pad,), jnp.int32)])
    dst = jnp.concatenate([e[1], jnp.full((pad,), N, jnp.int32)])
    return src.reshape(NS, NCH, CHUNK), dst.reshape(NS, NCH, CHUNK)


def kernel(x0, x1, e00, e01, e10, e11, W0, Wh, att):
    srcs, dsts = zip(*(_prep_edges(e) for e in (e00, e01, e10, e11)))
    z64 = jnp.zeros((RPT, H), jnp.float32)
    z16 = jnp.zeros((RPT, 16), jnp.float32)
    ones = jnp.ones((CHUNK, 16), jnp.float32)

    t = _tc0(x0, x1, W0)
    g = _sc_deg(*dsts, z16, ones)
    a = _sc_agg(*t, *srcs, *dsts, z64)
    h0, h1, *t = _tcmid(*a, *g, Wh[0])
    a = _sc_agg(*t, *srcs, *dsts, z64)
    q0, q1, *t = _tcmid(*a, *g, Wh[1])
    a = _sc_agg(*t, *srcs, *dsts, z64)
    p0, p1, *t = _tcmid(*a, *g, Wh[2])
    a = _sc_agg(*t, *srcs, *dsts, z64)
    _, _, *t = _tcmid(*a, *g, Wh[3])
    a = _sc_agg(*t, *srcs, *dsts, z64)
    return _tcfin(*a, *g, h0, h1, q0, q1, att)
